# baseline (speedup 1.0000x reference)
import functools

import jax
import jax.numpy as jnp
from jax import lax
from jax.experimental import pallas as pl
from jax.experimental.pallas import tpu as pltpu
from jax.experimental.pallas import tpu_sc as plsc

D_MODEL = 1024

NC = 2
NS = 16
NW = NC * NS

C = 48
NBUF = 2


@functools.lru_cache(maxsize=None)
def _build(n_chunks: int, d_model: int):
    b_per_w = n_chunks * C

    def body(tok_hbm, table_hbm, out_hbm, idx_v, buf_v, sems):
        wid = lax.axis_index("s") * NC + lax.axis_index("c")
        base = wid * b_per_w
        pltpu.sync_copy(tok_hbm.at[wid], idx_v)
        for b in range(NBUF):
            pltpu.async_copy(table_hbm.at[idx_v.at[b]], buf_v.at[b], sems.at[b])

        @pl.loop(0, n_chunks, step=NBUF)
        def _(c):
            for b in range(NBUF):
                cc = c + b
                pltpu.make_async_copy(
                    table_hbm.at[idx_v.at[cc]], buf_v.at[b], sems.at[b]
                ).wait()
                pltpu.sync_copy(
                    buf_v.at[b], out_hbm.at[pl.ds(base + cc * C, C)]
                )
                nxt = cc + NBUF

                @pl.when(nxt < n_chunks)
                def _():
                    pltpu.async_copy(
                        table_hbm.at[idx_v.at[nxt]], buf_v.at[b], sems.at[b]
                    )

    return pl.kernel(
        body,
        out_type=jax.ShapeDtypeStruct((NW * b_per_w, d_model), jnp.float32),
        mesh=plsc.VectorSubcoreMesh(core_axis_name="c", subcore_axis_name="s"),
        scratch_types=[
            pltpu.VMEM((n_chunks, C), jnp.int32),
            pltpu.VMEM((NBUF, C, d_model), jnp.float32),
            pltpu.SemaphoreType.DMA((NBUF,)),
        ],
    )


def kernel(tokens, embed_weights):
    n_tokens = tokens.size
    d_model = embed_weights.shape[1]
    grain = NW * C * NBUF
    n_pad = (-n_tokens) % grain
    tok_flat = tokens.reshape(-1).astype(jnp.int32)
    if n_pad:
        tok_flat = jnp.concatenate([tok_flat, jnp.zeros((n_pad,), jnp.int32)])
    n_chunks = tok_flat.size // (NW * C)
    tok3 = tok_flat.reshape(NW, n_chunks, C)
    out = _build(n_chunks, d_model)(tok3, embed_weights)
    if n_pad:
        out = out[:n_tokens]
    return out.reshape(tokens.shape + (d_model,))


# speedup vs baseline: 2.2397x; 2.2397x over previous
import functools

import jax
import jax.numpy as jnp
from jax import lax
from jax.experimental import pallas as pl
from jax.experimental.pallas import tpu as pltpu
from jax.experimental.pallas import tpu_sc as plsc

D_MODEL = 1024

NC = 2
NS = 16
NW = NC * NS

C = 32
NBUF = 2


@functools.lru_cache(maxsize=None)
def _build(n_chunks: int, d_model: int):
    b_per_w = n_chunks * C

    def body(tok_hbm, table_hbm, out_hbm, idx_v, buf_v, sems):
        wid = lax.axis_index("s") * NC + lax.axis_index("c")
        base = wid * b_per_w
        pltpu.sync_copy(tok_hbm.at[wid], idx_v)
        for b in range(NBUF):
            pltpu.async_copy(table_hbm.at[idx_v.at[b]], buf_v.at[b], sems.at[b])

        @pl.loop(0, n_chunks, step=NBUF)
        def _(c):
            for b in range(NBUF):
                cc = c + b
                pltpu.make_async_copy(
                    table_hbm.at[idx_v.at[cc]], buf_v.at[b], sems.at[b]
                ).wait()
                pltpu.sync_copy(
                    buf_v.at[b], out_hbm.at[pl.ds(base + cc * C, C)]
                )
                nxt = cc + NBUF

                @pl.when(nxt < n_chunks)
                def _():
                    pltpu.async_copy(
                        table_hbm.at[idx_v.at[nxt]], buf_v.at[b], sems.at[b]
                    )

    return pl.kernel(
        body,
        out_type=jax.ShapeDtypeStruct((NW * b_per_w, d_model), jnp.float32),
        mesh=plsc.VectorSubcoreMesh(core_axis_name="c", subcore_axis_name="s"),
        scratch_types=[
            pltpu.VMEM((n_chunks, C), jnp.int32),
            pltpu.VMEM((NBUF, C, d_model), jnp.float32),
            pltpu.SemaphoreType.DMA((NBUF,)),
        ],
    )


def kernel(tokens, embed_weights):
    n_tokens = tokens.size
    d_model = embed_weights.shape[1]
    grain = NW * C * NBUF
    n_pad = (-n_tokens) % grain
    tok_flat = tokens.reshape(-1).astype(jnp.int32)
    if n_pad:
        tok_flat = jnp.concatenate([tok_flat, jnp.zeros((n_pad,), jnp.int32)])
    n_chunks = tok_flat.size // (NW * C)
    tok3 = tok_flat.reshape(NW, n_chunks, C)
    out = _build(n_chunks, d_model)(tok3, embed_weights)
    if n_pad:
        out = out[:n_tokens]
    return out.reshape(tokens.shape + (d_model,))
